# baseline (speedup 1.0000x reference)
import functools

import jax
import jax.numpy as jnp
from jax import lax
from jax.experimental import pallas as pl
from jax.experimental.pallas import tpu as pltpu
from jax.experimental.pallas import tpu_sc as plsc

N = 10000
NPAD = 10240
E = 320000
D = 128
K = 3

NC = 2
NS = 16
NW = NC * NS
EPW = E // NW
CB = 400
CC = 80
RPT = NPAD // NS

BN = 1024
BE = 1280
BND = 1024

_LANES = 16


def _dense_body(x_ref, w_ref, att_ref, h_ref, s1_ref, s2_ref):
    s1_cols, s2_cols = [], []
    for k in range(K):
        xw = jnp.dot(x_ref[...], w_ref[k], preferred_element_type=jnp.float32)
        h_ref[k] = xw
        a1 = att_ref[:D, k:k + 1]
        a2 = att_ref[D:2 * D, k:k + 1]
        s1_cols.append(jnp.dot(xw, a1, preferred_element_type=jnp.float32))
        s2_cols.append(jnp.dot(xw, a2, preferred_element_type=jnp.float32))
    s1_ref[...] = jnp.concatenate(s1_cols, axis=1)
    s2_ref[...] = jnp.concatenate(s2_cols, axis=1)


def _dense_phase(xp, W, att_t):
    return pl.pallas_call(
        _dense_body,
        grid=(NPAD // BN,),
        in_specs=[
            pl.BlockSpec((BN, D), lambda i: (i, 0)),
            pl.BlockSpec((K, D, D), lambda i: (0, 0, 0)),
            pl.BlockSpec((3 * D, K), lambda i: (0, 0)),
        ],
        out_specs=[
            pl.BlockSpec((K, BN, D), lambda i: (0, i, 0)),
            pl.BlockSpec((BN, K), lambda i: (i, 0)),
            pl.BlockSpec((BN, K), lambda i: (i, 0)),
        ],
        out_shape=[
            jax.ShapeDtypeStruct((K, NPAD, D), jnp.float32),
            jax.ShapeDtypeStruct((NPAD, K), jnp.float32),
            jax.ShapeDtypeStruct((NPAD, K), jnp.float32),
        ],
    )(xp, W, att_t)


def _emat_body(ea_ref, we_ref, att_ref, em_ref):
    rows = []
    for k in range(K):
        a3 = att_ref[2 * D:, k:k + 1]
        ce = jnp.dot(we_ref[k], a3, preferred_element_type=jnp.float32)
        rows.append(lax.dot_general(
            ce, ea_ref[...], (((0,), (1,)), ((), ())),
            preferred_element_type=jnp.float32))
    em_ref[...] = jnp.concatenate(rows, axis=0)


def _emat_phase(edge_attr, We, att_t):
    return pl.pallas_call(
        _emat_body,
        grid=(E // BE,),
        in_specs=[
            pl.BlockSpec((BE, 16), lambda i: (i, 0)),
            pl.BlockSpec((K, 16, D), lambda i: (0, 0, 0)),
            pl.BlockSpec((3 * D, K), lambda i: (0, 0)),
        ],
        out_specs=pl.BlockSpec((K, BE), lambda i: (0, i)),
        out_shape=jax.ShapeDtypeStruct((K, E), jnp.float32),
    )(edge_attr, We, att_t)


def _alpha_body(lg_ref, al_ref):
    al_ref[...] = jnp.transpose(lg_ref[...], (1, 0))


def _alpha_phase(lg2):
    return pl.pallas_call(
        _alpha_body,
        grid=(E // BE,),
        in_specs=[pl.BlockSpec((K, BE), lambda i: (0, i))],
        out_specs=pl.BlockSpec((BE, K), lambda i: (i, 0)),
        out_shape=jax.ShapeDtypeStruct((E, K), jnp.float32),
    )(lg2)


_MESH = plsc.VectorSubcoreMesh(core_axis_name="c", subcore_axis_name="s")
_DEN_ROWS = K * NPAD // _LANES


@functools.partial(
    pl.kernel,
    out_type=[
        jax.ShapeDtypeStruct((K * E,), jnp.float32),
        jax.ShapeDtypeStruct((K * E,), jnp.float32),
        jax.ShapeDtypeStruct((NW * K * NPAD,), jnp.float32),
    ],
    mesh=_MESH,
    scratch_types=[
        pltpu.VMEM((NPAD * K,), jnp.float32),
        pltpu.VMEM((NPAD * K,), jnp.float32),
        pltpu.VMEM((NPAD * K,), jnp.float32),
        pltpu.VMEM((CB,), jnp.int32),
        pltpu.VMEM((CB,), jnp.int32),
        pltpu.VMEM((CB * K,), jnp.float32),
        pltpu.VMEM((CB * K,), jnp.float32),
        pltpu.VMEM((CB * K,), jnp.float32),
    ],
    compiler_params=pltpu.CompilerParams(needs_layout_passes=False),
)
def _phase_b(s1_hbm, s2_hbm, em_hbm, src_hbm, dst_hbm,
             logit_out, ex_out, den_out,
             s1_v, s2_v, den_v, src_v, dst_v, em_v, lg_v, ex_v):
    cid = lax.axis_index("c")
    sid = lax.axis_index("s")
    wid = sid * NC + cid

    pltpu.sync_copy(s1_hbm, s1_v)
    pltpu.sync_copy(s2_hbm, s2_v)

    def _zero(i, _):
        den_v[pl.ds(i * _LANES, _LANES)] = jnp.zeros((_LANES,), jnp.float32)
        return 0
    lax.fori_loop(0, NPAD * K // _LANES, _zero, 0)

    ebase = wid * EPW

    def _chunk(c, _):
        base = ebase + c * CB
        pltpu.sync_copy(src_hbm.at[pl.ds(base, CB)], src_v)
        pltpu.sync_copy(dst_hbm.at[pl.ds(base, CB)], dst_v)
        for k in range(K):
            pltpu.sync_copy(em_hbm.at[pl.ds(k * E + base, CB)],
                            em_v.at[pl.ds(k * CB, CB)])

        def _vstep(j, _):
            sv = src_v[pl.ds(j * _LANES, _LANES)] * K
            dv = dst_v[pl.ds(j * _LANES, _LANES)] * K
            for k in range(K):
                g1 = plsc.load_gather(s1_v, [sv + k])
                g2 = plsc.load_gather(s2_v, [dv + k])
                em = em_v[pl.ds(k * CB + j * _LANES, _LANES)]
                l = g1 + g2 + em
                l = jnp.where(l >= 0.0, l, l * jnp.float32(0.2))
                ex = jnp.exp(l)
                lg_v[pl.ds(k * CB + j * _LANES, _LANES)] = l
                ex_v[pl.ds(k * CB + j * _LANES, _LANES)] = ex
                plsc.addupdate_scatter(den_v, [dv + k], ex)
            return 0
        lax.fori_loop(0, CB // _LANES, _vstep, 0)

        for k in range(K):
            pltpu.sync_copy(lg_v.at[pl.ds(k * CB, CB)],
                            logit_out.at[pl.ds(k * E + base, CB)])
            pltpu.sync_copy(ex_v.at[pl.ds(k * CB, CB)],
                            ex_out.at[pl.ds(k * E + base, CB)])
        return 0
    lax.fori_loop(0, EPW // CB, _chunk, 0)

    pltpu.sync_copy(den_v, den_out.at[pl.ds(wid * K * NPAD, K * NPAD)])


NCH = EPW // CC
NB = 3


@functools.partial(
    pl.kernel,
    out_type=jax.ShapeDtypeStruct((NC * K * NPAD, D), jnp.float32),
    mesh=_MESH,
    scratch_types=[
        pltpu.VMEM((EPW,), jnp.int32),
        [pltpu.VMEM((CC,), jnp.int32) for _ in range(NB)],
        [pltpu.VMEM((CC,), jnp.int32) for _ in range(NB)],
        [pltpu.VMEM((CC,), jnp.float32) for _ in range(NB)],
        [pltpu.VMEM((CC, D), jnp.float32) for _ in range(NB)],
        pltpu.VMEM_SHARED((NPAD, D), jnp.float32),
        [pltpu.SemaphoreType.DMA for _ in range(NB)],
        [pltpu.SemaphoreType.DMA for _ in range(NB)],
        [pltpu.SemaphoreType.DMA for _ in range(NB)],
    ],
    compiler_params=pltpu.CompilerParams(needs_layout_passes=False),
)
def _phase_c(h_hbm, src_hbm, dst_hbm, ex_hbm, acc_out,
             src_v, idxs, dbs, exbs, rows, acc_sh, gsems, ssems, psems):
    cid = lax.axis_index("c")
    sid = lax.axis_index("s")
    wid = sid * NC + cid
    ebase = wid * EPW

    pltpu.sync_copy(src_hbm.at[pl.ds(ebase, EPW)], src_v)

    def _mkidx(c, k, ib):
        for j in range(CC // _LANES):
            ib[pl.ds(j * _LANES, _LANES)] = (
                src_v[pl.ds(c * CC + j * _LANES, _LANES)] + k * NPAD)

    def _scale(eb, rb):
        def _sj(jb, _):
            w16 = eb[pl.ds(jb * _LANES, _LANES)]
            for i in range(_LANES):
                w = jnp.full((_LANES,), w16[i], jnp.float32)
                r = jb * _LANES + i
                for j in range(D // _LANES):
                    rb[r, pl.ds(j * _LANES, _LANES)] = (
                        rb[r, pl.ds(j * _LANES, _LANES)] * w)
            return 0
        lax.fori_loop(0, CC // _LANES, _sj, 0)

    def _prefetch(c, k, u):
        pltpu.async_copy(dst_hbm.at[pl.ds(ebase + c * CC, CC)], dbs[u],
                         psems[u])
        pltpu.async_copy(ex_hbm.at[pl.ds(k * E + ebase + c * CC, CC)],
                         exbs[u], psems[u])

    def _wait_prefetch(c, k, u):
        pltpu.make_async_copy(dst_hbm.at[pl.ds(ebase + c * CC, CC)], dbs[u],
                              psems[u]).wait()
        pltpu.make_async_copy(ex_hbm.at[pl.ds(k * E + ebase + c * CC, CC)],
                              exbs[u], psems[u]).wait()

    def _issue_gather(c, k, u):
        _mkidx(c, k, idxs[u])
        pltpu.async_copy(h_hbm.at[idxs[u]], rows[u], gsems[u])

    def _wait_scatter(u):
        pltpu.make_async_copy(rows[u], acc_sh.at[dbs[u]], ssems[u]).wait()

    def _finish(c, k, u):
        pltpu.make_async_copy(h_hbm.at[idxs[u]], rows[u], gsems[u]).wait()
        _wait_prefetch(c, k, u)
        _scale(exbs[u], rows[u])
        pltpu.async_copy(rows[u], acc_sh.at[dbs[u]], ssems[u], add=True)

    for k in range(K):
        def _zb(r, _):
            for j in range(D // _LANES):
                rows[0][r, pl.ds(j * _LANES, _LANES)] = (
                    jnp.zeros((_LANES,), jnp.float32))
            return 0
        lax.fori_loop(0, CC, _zb, 0)
        for i in range(RPT // CC):
            pltpu.sync_copy(rows[0], acc_sh.at[pl.ds(sid * RPT + i * CC, CC)])
        plsc.subcore_barrier()

        _prefetch(0, k, 0)
        _issue_gather(0, k, 0)

        def _triple(i, _):
            for u in range(NB):
                c = i * NB + u
                u1 = (u + 1) % NB

                @pl.when(c >= 2)
                def _():
                    _wait_scatter(u1)

                @pl.when(c + 1 < NCH)
                def _():
                    _prefetch(c + 1, k, u1)
                    _issue_gather(c + 1, k, u1)
                _finish(c, k, u)
            return 0
        lax.fori_loop(0, NCH // NB, _triple, 0)
        for c in range(NCH - NCH % NB, NCH):
            u = c % NB
            u1 = (u + 1) % NB
            _wait_scatter(u1)
            if c + 1 < NCH:
                _prefetch(c + 1, k, u1)
                _issue_gather(c + 1, k, u1)
            _finish(c, k, u)
        _wait_scatter((NCH - 2) % NB)
        _wait_scatter((NCH - 1) % NB)

        plsc.subcore_barrier()
        pltpu.sync_copy(
            acc_sh.at[pl.ds(sid * RPT, RPT)],
            acc_out.at[pl.ds(cid * (K * NPAD) + k * NPAD + sid * RPT, RPT)])
        plsc.subcore_barrier()


_BR = 3072


def _denred_body(d_ref, o_ref):
    o_ref[...] = jnp.sum(d_ref[...], axis=0)


def _denred_phase(den2):
    return pl.pallas_call(
        _denred_body,
        grid=(NPAD * K // _BR,),
        in_specs=[pl.BlockSpec((NW, _BR), lambda i: (0, i))],
        out_specs=pl.BlockSpec((_BR,), lambda i: (i,)),
        out_shape=jax.ShapeDtypeStruct((NPAD * K,), jnp.float32),
    )(den2)


def _combine_body(acc_ref, den_ref, o_ref):
    acc = acc_ref[0] + acc_ref[1]
    o = jnp.zeros((BND, D), jnp.float32)
    for k in range(K):
        v = acc[k] / (den_ref[:, k:k + 1] + jnp.float32(1e-16))
        o = o + jnp.where(v >= 0.0, v, v * jnp.float32(0.01))
    o_ref[...] = o


def _combine_phase(acc4, den3):
    return pl.pallas_call(
        _combine_body,
        grid=(NPAD // BND,),
        in_specs=[
            pl.BlockSpec((NC, K, BND, D), lambda i: (0, 0, i, 0)),
            pl.BlockSpec((BND, K), lambda i: (i, 0)),
        ],
        out_specs=pl.BlockSpec((BND, D), lambda i: (i, 0)),
        out_shape=jax.ShapeDtypeStruct((NPAD, D), jnp.float32),
    )(acc4, den3)


def kernel(x, edge_attr, edge_index, W, We, att):
    src = edge_index[0]
    dst = edge_index[1]
    att_t = att.T
    xp = jnp.pad(x, ((0, NPAD - N), (0, 0)))

    h, s1, s2 = _dense_phase(xp, W, att_t)
    em = _emat_phase(edge_attr, We, att_t)

    logit_flat, ex_flat, den_flat = _phase_b(
        s1.reshape(NPAD * K), s2.reshape(NPAD * K), em.reshape(K * E),
        src, dst)

    acc = _phase_c(h.reshape(K * NPAD, D), src, dst, ex_flat)

    acc4 = acc.reshape(NC, K, NPAD, D)
    den_tot = _denred_phase(den_flat.reshape(NW, NPAD * K))
    emb = _combine_phase(acc4, den_tot.reshape(NPAD, K))

    alphas = _alpha_phase(logit_flat.reshape(K, E))
    return emb[:N], alphas


# speedup vs baseline: 1.2496x; 1.2496x over previous
import functools

import jax
import jax.numpy as jnp
from jax import lax
from jax.experimental import pallas as pl
from jax.experimental.pallas import tpu as pltpu
from jax.experimental.pallas import tpu_sc as plsc

N = 10000
NPAD = 10240
E = 320000
D = 128
K = 3

NC = 2
NS = 16
NW = NC * NS
EPW = E // NW
CB = 400
CC = 80
RPT = NPAD // NS

BN = 1024
BE = 1280
BND = 1024

_LANES = 16


def _dense_body(x_ref, w_ref, att_ref, h_ref, s1_ref, s2_ref):
    s1_cols, s2_cols = [], []
    for k in range(K):
        xw = jnp.dot(x_ref[...], w_ref[k], preferred_element_type=jnp.float32)
        h_ref[k] = xw
        a1 = att_ref[:D, k:k + 1]
        a2 = att_ref[D:2 * D, k:k + 1]
        s1_cols.append(jnp.dot(xw, a1, preferred_element_type=jnp.float32))
        s2_cols.append(jnp.dot(xw, a2, preferred_element_type=jnp.float32))
    s1_ref[...] = jnp.concatenate(s1_cols, axis=1)
    s2_ref[...] = jnp.concatenate(s2_cols, axis=1)


def _dense_phase(xp, W, att_t):
    return pl.pallas_call(
        _dense_body,
        grid=(NPAD // BN,),
        in_specs=[
            pl.BlockSpec((BN, D), lambda i: (i, 0)),
            pl.BlockSpec((K, D, D), lambda i: (0, 0, 0)),
            pl.BlockSpec((3 * D, K), lambda i: (0, 0)),
        ],
        out_specs=[
            pl.BlockSpec((K, BN, D), lambda i: (0, i, 0)),
            pl.BlockSpec((BN, K), lambda i: (i, 0)),
            pl.BlockSpec((BN, K), lambda i: (i, 0)),
        ],
        out_shape=[
            jax.ShapeDtypeStruct((K, NPAD, D), jnp.float32),
            jax.ShapeDtypeStruct((NPAD, K), jnp.float32),
            jax.ShapeDtypeStruct((NPAD, K), jnp.float32),
        ],
    )(xp, W, att_t)


EA8 = E // 8
BE8 = 1024


EPW8 = EPW // 8


def _emat_body(ea_ref, we_ref, att_ref, em_ref):
    gidx = lax.broadcasted_iota(jnp.int32, (D, 8), 0) // 16
    gcol = lax.broadcasted_iota(jnp.int32, (D, 8), 1)
    rows = []
    for k in range(K):
        a3 = att_ref[2 * D:, k:k + 1]
        ce = jnp.dot(we_ref[k], a3, preferred_element_type=jnp.float32)
        cet = jnp.concatenate([ce] * 8, axis=0)
        cebd = jnp.where(gidx == gcol, cet, 0.0)
        rows.append(lax.dot_general(
            cebd, ea_ref[0], (((0,), (1,)), ((), ())),
            preferred_element_type=jnp.float32))
    em_ref[0] = jnp.concatenate(rows, axis=0)


def _emat_phase(ea8, We, att_t):
    return pl.pallas_call(
        _emat_body,
        grid=(NW,),
        in_specs=[
            pl.BlockSpec((1, EPW8, D), lambda i: (i, 0, 0)),
            pl.BlockSpec((K, 16, D), lambda i: (0, 0, 0)),
            pl.BlockSpec((3 * D, K), lambda i: (0, 0)),
        ],
        out_specs=pl.BlockSpec((1, K * 8, EPW8), lambda i: (i, 0, 0)),
        out_shape=jax.ShapeDtypeStruct((NW, K * 8, EPW8), jnp.float32),
    )(ea8, We, att_t)


_MESH = plsc.VectorSubcoreMesh(core_axis_name="c", subcore_axis_name="s")
_DEN_ROWS = K * NPAD // _LANES


@functools.partial(
    pl.kernel,
    out_type=[
        jax.ShapeDtypeStruct((K * E,), jnp.float32),
        jax.ShapeDtypeStruct((K * E,), jnp.float32),
        jax.ShapeDtypeStruct((NW * K * NPAD,), jnp.float32),
    ],
    mesh=_MESH,
    scratch_types=[
        pltpu.VMEM((NPAD * K,), jnp.float32),
        pltpu.VMEM((NPAD * K,), jnp.float32),
        pltpu.VMEM((NPAD * K,), jnp.float32),
        pltpu.VMEM((CB,), jnp.int32),
        pltpu.VMEM((CB,), jnp.int32),
        pltpu.VMEM((K * 8, EPW8), jnp.float32),
        pltpu.VMEM((CB * K,), jnp.float32),
        pltpu.VMEM((CB * K,), jnp.float32),
    ],
    compiler_params=pltpu.CompilerParams(needs_layout_passes=False),
)
def _phase_b(s1_hbm, s2_hbm, em_hbm, src_hbm, dst_hbm,
             logit_out, ex_out, den_out,
             s1_v, s2_v, den_v, src_v, dst_v, em_v, lg_v, ex_v):
    cid = lax.axis_index("c")
    sid = lax.axis_index("s")
    wid = sid * NC + cid

    pltpu.sync_copy(s1_hbm, s1_v)
    pltpu.sync_copy(s2_hbm, s2_v)
    pltpu.sync_copy(em_hbm.at[wid], em_v)

    def _zero(i, _):
        den_v[pl.ds(i * _LANES, _LANES)] = jnp.zeros((_LANES,), jnp.float32)
        return 0
    lax.fori_loop(0, NPAD * K // _LANES, _zero, 0)

    ebase = wid * EPW

    def _chunk(c, _):
        base = ebase + c * CB
        pltpu.sync_copy(src_hbm.at[pl.ds(base, CB)], src_v)
        pltpu.sync_copy(dst_hbm.at[pl.ds(base, CB)], dst_v)

        def _vstep(j, _):
            sv = src_v[pl.ds(j * _LANES, _LANES)] * K
            dv = dst_v[pl.ds(j * _LANES, _LANES)] * K
            el = lax.iota(jnp.int32, _LANES) + (c * CB + j * _LANES)
            erow = lax.bitwise_and(el, 7)
            ecol = lax.shift_right_logical(el, 3)
            for k in range(K):
                g1 = plsc.load_gather(s1_v, [sv + k])
                g2 = plsc.load_gather(s2_v, [dv + k])
                em = plsc.load_gather(em_v, [erow + k * 8, ecol])
                l = g1 + g2 + em
                l = jnp.where(l >= 0.0, l, l * jnp.float32(0.2))
                ex = jnp.exp(l)
                lg_v[pl.ds(k * CB + j * _LANES, _LANES)] = l
                ex_v[pl.ds(k * CB + j * _LANES, _LANES)] = ex
                plsc.addupdate_scatter(den_v, [dv + k], ex)
            return 0
        lax.fori_loop(0, CB // _LANES, _vstep, 0)

        for k in range(K):
            pltpu.sync_copy(lg_v.at[pl.ds(k * CB, CB)],
                            logit_out.at[pl.ds(k * E + base, CB)])
            pltpu.sync_copy(ex_v.at[pl.ds(k * CB, CB)],
                            ex_out.at[pl.ds(k * E + base, CB)])
        return 0
    lax.fori_loop(0, EPW // CB, _chunk, 0)

    pltpu.sync_copy(den_v, den_out.at[pl.ds(wid * K * NPAD, K * NPAD)])


NCH = EPW // CC
NB = 3


@functools.partial(
    pl.kernel,
    out_type=jax.ShapeDtypeStruct((NC * K * NPAD, D), jnp.float32),
    mesh=_MESH,
    scratch_types=[
        pltpu.VMEM((EPW,), jnp.int32),
        [pltpu.VMEM((CC,), jnp.int32) for _ in range(NB)],
        [pltpu.VMEM((CC,), jnp.int32) for _ in range(NB)],
        [pltpu.VMEM((CC,), jnp.float32) for _ in range(NB)],
        [pltpu.VMEM((CC, D), jnp.float32) for _ in range(NB)],
        pltpu.VMEM_SHARED((NPAD, D), jnp.float32),
        [pltpu.SemaphoreType.DMA for _ in range(NB)],
        [pltpu.SemaphoreType.DMA for _ in range(NB)],
        [pltpu.SemaphoreType.DMA for _ in range(NB)],
    ],
    compiler_params=pltpu.CompilerParams(needs_layout_passes=False),
)
def _phase_c(h_hbm, src_hbm, dst_hbm, ex_hbm, acc_out,
             src_v, idxs, dbs, exbs, rows, acc_sh, gsems, ssems, psems):
    cid = lax.axis_index("c")
    sid = lax.axis_index("s")
    wid = sid * NC + cid
    ebase = wid * EPW

    pltpu.sync_copy(src_hbm.at[pl.ds(ebase, EPW)], src_v)

    def _mkidx(c, k, ib):
        for j in range(CC // _LANES):
            ib[pl.ds(j * _LANES, _LANES)] = (
                src_v[pl.ds(c * CC + j * _LANES, _LANES)] + k * NPAD)

    def _scale(eb, rb):
        def _sj(jb, _):
            w16 = eb[pl.ds(jb * _LANES, _LANES)]
            for i in range(_LANES):
                w = jnp.full((_LANES,), w16[i], jnp.float32)
                r = jb * _LANES + i
                for j in range(D // _LANES):
                    rb[r, pl.ds(j * _LANES, _LANES)] = (
                        rb[r, pl.ds(j * _LANES, _LANES)] * w)
            return 0
        lax.fori_loop(0, CC // _LANES, _sj, 0)

    def _prefetch(c, k, u):
        pltpu.async_copy(dst_hbm.at[pl.ds(ebase + c * CC, CC)], dbs[u],
                         psems[u])
        pltpu.async_copy(ex_hbm.at[pl.ds(k * E + ebase + c * CC, CC)],
                         exbs[u], psems[u])

    def _wait_prefetch(c, k, u):
        pltpu.make_async_copy(dst_hbm.at[pl.ds(ebase + c * CC, CC)], dbs[u],
                              psems[u]).wait()
        pltpu.make_async_copy(ex_hbm.at[pl.ds(k * E + ebase + c * CC, CC)],
                              exbs[u], psems[u]).wait()

    def _issue_gather(c, k, u):
        _mkidx(c, k, idxs[u])
        pltpu.async_copy(h_hbm.at[idxs[u]], rows[u], gsems[u])

    def _wait_scatter(u):
        pltpu.make_async_copy(rows[u], acc_sh.at[dbs[u]], ssems[u]).wait()

    def _finish(c, k, u):
        pltpu.make_async_copy(h_hbm.at[idxs[u]], rows[u], gsems[u]).wait()
        _wait_prefetch(c, k, u)
        _scale(exbs[u], rows[u])
        pltpu.async_copy(rows[u], acc_sh.at[dbs[u]], ssems[u], add=True)

    for k in range(K):
        def _zb(r, _):
            for j in range(D // _LANES):
                rows[0][r, pl.ds(j * _LANES, _LANES)] = (
                    jnp.zeros((_LANES,), jnp.float32))
            return 0
        lax.fori_loop(0, CC, _zb, 0)
        for i in range(RPT // CC):
            pltpu.sync_copy(rows[0], acc_sh.at[pl.ds(sid * RPT + i * CC, CC)])
        plsc.subcore_barrier()

        _prefetch(0, k, 0)
        _issue_gather(0, k, 0)

        def _triple(i, _):
            for u in range(NB):
                c = i * NB + u
                u1 = (u + 1) % NB

                @pl.when(c >= 2)
                def _():
                    _wait_scatter(u1)

                @pl.when(c + 1 < NCH)
                def _():
                    _prefetch(c + 1, k, u1)
                    _issue_gather(c + 1, k, u1)
                _finish(c, k, u)
            return 0
        lax.fori_loop(0, NCH // NB, _triple, 0)
        for c in range(NCH - NCH % NB, NCH):
            u = c % NB
            u1 = (u + 1) % NB
            _wait_scatter(u1)
            if c + 1 < NCH:
                _prefetch(c + 1, k, u1)
                _issue_gather(c + 1, k, u1)
            _finish(c, k, u)
        _wait_scatter((NCH - 2) % NB)
        _wait_scatter((NCH - 1) % NB)

        plsc.subcore_barrier()
        pltpu.sync_copy(
            acc_sh.at[pl.ds(sid * RPT, RPT)],
            acc_out.at[pl.ds(cid * (K * NPAD) + k * NPAD + sid * RPT, RPT)])
        plsc.subcore_barrier()


_BR = 3072


def _denred_body(d_ref, o_ref):
    o_ref[...] = jnp.sum(d_ref[...], axis=0)


def _denred_phase(den2):
    return pl.pallas_call(
        _denred_body,
        grid=(NPAD * K // _BR,),
        in_specs=[pl.BlockSpec((NW, _BR), lambda i: (0, i))],
        out_specs=pl.BlockSpec((_BR,), lambda i: (i,)),
        out_shape=jax.ShapeDtypeStruct((NPAD * K,), jnp.float32),
    )(den2)


def _combine_body(acc_ref, den_ref, o_ref):
    acc = acc_ref[0] + acc_ref[1]
    o = jnp.zeros((BND, D), jnp.float32)
    for k in range(K):
        v = acc[k] / (den_ref[:, k:k + 1] + jnp.float32(1e-16))
        o = o + jnp.where(v >= 0.0, v, v * jnp.float32(0.01))
    o_ref[...] = o


def _combine_phase(acc4, den3):
    return pl.pallas_call(
        _combine_body,
        grid=(NPAD // BND,),
        in_specs=[
            pl.BlockSpec((NC, K, BND, D), lambda i: (0, 0, i, 0)),
            pl.BlockSpec((BND, K), lambda i: (i, 0)),
        ],
        out_specs=pl.BlockSpec((BND, D), lambda i: (i, 0)),
        out_shape=jax.ShapeDtypeStruct((NPAD, D), jnp.float32),
    )(acc4, den3)


def kernel(x, edge_attr, edge_index, W, We, att):
    src = edge_index[0]
    dst = edge_index[1]
    att_t = att.T
    xp = jnp.pad(x, ((0, NPAD - N), (0, 0)))

    h, s1, s2 = _dense_phase(xp, W, att_t)
    em = _emat_phase(edge_attr.reshape(NW, EPW8, D), We, att_t)

    logit_flat, ex_flat, den_flat = _phase_b(
        s1.reshape(NPAD * K), s2.reshape(NPAD * K), em, src, dst)

    acc = _phase_c(h.reshape(K * NPAD, D), src, dst, ex_flat)

    acc4 = acc.reshape(NC, K, NPAD, D)
    den_tot = _denred_phase(den_flat.reshape(NW, NPAD * K))
    emb = _combine_phase(acc4, den_tot.reshape(NPAD, K))

    alphas = jnp.transpose(logit_flat.reshape(K, E), (1, 0))
    return emb[:N], alphas
